# baseline (speedup 1.0000x reference)
import functools

import jax
import jax.numpy as jnp
from jax import lax
from jax.experimental import pallas as pl
from jax.experimental.pallas import tpu as pltpu
from jax.experimental.pallas import tpu_sc as plsc

_NUM_CODES = 8192
_CODE_DIM = 256
_SPLITS = 2
_COMMITMENT = 0.25

_BM = 4096
_BN = 1024
_NB = _NUM_CODES // _BN

_NC = 2
_NS = 16
_NW = _NC * _NS
_GCH = 128


def _argmin_body(flat_ref, emb_ref, an_ref, en_ref, idx_ref, loss_ref):
    m = pl.program_id(0)
    a = an_ref[...]
    flat = flat_ref[...]

    ids = lax.broadcasted_iota(jnp.int32, (_BM, _BN), 1).astype(jnp.float32)
    run_min = jnp.full((_BM, 1), jnp.inf, jnp.float32)
    run_arg = jnp.zeros((_BM, 1), jnp.float32)
    for i in range(_NB):
        eblk = emb_ref[pl.ds(i * _BN, _BN), :]
        b = lax.dot_general(flat, eblk, (((1,), (1,)), ((), ())),
                            preferred_element_type=jnp.float32)
        c = en_ref[:, pl.ds(i * _BN, _BN)]
        dist = (a - 2.0 * b) + c
        lmin = jnp.min(dist, axis=1, keepdims=True)
        larg = jnp.min(jnp.where(dist == lmin, ids, jnp.float32(2**24)),
                       axis=1, keepdims=True) + jnp.float32(i * _BN)
        upd = lmin < run_min
        run_min = jnp.where(upd, lmin, run_min)
        run_arg = jnp.where(upd, larg, run_arg)
    idx_ref[...] = run_arg.astype(jnp.int32)
    s = jnp.sum(run_min, keepdims=True)
    prev = jnp.where(m == 0, jnp.zeros((1, 1), jnp.float32), loss_ref[...])
    loss_ref[...] = prev + s


def _tc_argmin(flat, embedding, a_norms, e_norms):
    m_total = flat.shape[0]
    return pl.pallas_call(
        _argmin_body,
        grid=(m_total // _BM,),
        in_specs=[
            pl.BlockSpec((_BM, _CODE_DIM), lambda m: (m, 0)),
            pl.BlockSpec((_NUM_CODES, _CODE_DIM), lambda m: (0, 0)),
            pl.BlockSpec((_BM, 1), lambda m: (m, 0)),
            pl.BlockSpec((1, _NUM_CODES), lambda m: (0, 0)),
        ],
        out_specs=[
            pl.BlockSpec((_BM, 1), lambda m: (m, 0)),
            pl.BlockSpec((1, 1), lambda m: (0, 0)),
        ],
        out_shape=[
            jax.ShapeDtypeStruct((m_total, 1), jnp.int32),
            jax.ShapeDtypeStruct((1, 1), jnp.float32),
        ],
    )(flat, embedding, a_norms, e_norms)


def _make_sc_gather(b_total):
    bpw = b_total // _NW
    nch = bpw // _GCH
    mesh = plsc.VectorSubcoreMesh(core_axis_name="c", subcore_axis_name="s")

    @functools.partial(
        pl.kernel, mesh=mesh,
        out_type=jax.ShapeDtypeStruct((b_total, _CODE_DIM), jnp.float32),
        scratch_types=[
            pltpu.VMEM((bpw,), jnp.int32),
            pltpu.VMEM((2, _GCH, _CODE_DIM), jnp.float32),
            pltpu.SemaphoreType.DMA,
            pltpu.SemaphoreType.DMA,
        ],
    )
    def gather(table_hbm, idx_hbm, out_hbm, idx_v, rows_v, sem0, sem1):
        wid = lax.axis_index("s") * _NC + lax.axis_index("c")
        base = wid * bpw
        pltpu.sync_copy(idx_hbm.at[pl.ds(base, bpw)], idx_v)
        sems = (sem0, sem1)
        copies = [None, None]
        for c in range(nch):
            sl = c % 2
            copies[sl] = pltpu.async_copy(
                table_hbm.at[idx_v.at[pl.ds(c * _GCH, _GCH)]],
                rows_v.at[sl], sems[sl])
            if c > 0:
                copies[(c - 1) % 2].wait()
                pltpu.sync_copy(rows_v.at[(c - 1) % 2],
                                out_hbm.at[pl.ds(base + (c - 1) * _GCH, _GCH)])
        copies[(nch - 1) % 2].wait()
        pltpu.sync_copy(rows_v.at[(nch - 1) % 2],
                        out_hbm.at[pl.ds(base + (nch - 1) * _GCH, _GCH)])

    return gather


def kernel(z, embedding):
    orig_shape = z.shape
    flat = z.reshape(-1, _CODE_DIM)
    m_total = flat.shape[0]
    a_norms = jnp.sum(flat * flat, axis=1, keepdims=True)
    e_norms = jnp.sum(embedding * embedding, axis=1).reshape(1, -1)
    idx2d, loss_sum = _tc_argmin(flat, embedding, a_norms, e_norms)
    indices = idx2d.reshape(-1)
    z_q_rows = _make_sc_gather(m_total)(embedding, indices)
    z_q_flat = z_q_rows.reshape(orig_shape)
    loss = (_COMMITMENT / (m_total * _CODE_DIM)) * loss_sum[0, 0]
    indices_out = indices.reshape(orig_shape[:-1] + (_SPLITS,))
    return (z_q_flat, loss, indices_out)


# speedup vs baseline: 1.1348x; 1.1348x over previous
import functools

import jax
import jax.numpy as jnp
from jax import lax
from jax.experimental import pallas as pl
from jax.experimental.pallas import tpu as pltpu
from jax.experimental.pallas import tpu_sc as plsc

_NUM_CODES = 8192
_CODE_DIM = 256
_SPLITS = 2
_COMMITMENT = 0.25

_BM = 2048
_BN = 512
_NB = _NUM_CODES // _BN

_NC = 2
_NS = 16
_NW = _NC * _NS
_GCH = 128


def _argmin_body(flat_ref, emb_ref, an_ref, en_ref, idx_ref, loss_ref):
    m = pl.program_id(0)
    a = an_ref[...]
    flat = flat_ref[...]

    ids = lax.broadcasted_iota(jnp.int32, (_BM, _BN), 1).astype(jnp.float32)
    run_min = jnp.full((_BM, 1), jnp.inf, jnp.float32)
    run_arg = jnp.zeros((_BM, 1), jnp.float32)
    for i in range(_NB):
        eblk = emb_ref[pl.ds(i * _BN, _BN), :]
        b = lax.dot_general(flat, eblk, (((1,), (1,)), ((), ())),
                            preferred_element_type=jnp.float32)
        c = en_ref[:, pl.ds(i * _BN, _BN)]
        dist = (a - 2.0 * b) + c
        lmin = jnp.min(dist, axis=1, keepdims=True)
        larg = jnp.min(jnp.where(dist == lmin, ids, jnp.float32(2**24)),
                       axis=1, keepdims=True) + jnp.float32(i * _BN)
        upd = lmin < run_min
        run_min = jnp.where(upd, lmin, run_min)
        run_arg = jnp.where(upd, larg, run_arg)
    idx_ref[...] = run_arg.astype(jnp.int32)
    s = jnp.sum(run_min, keepdims=True)
    prev = jnp.where(m == 0, jnp.zeros((1, 1), jnp.float32), loss_ref[...])
    loss_ref[...] = prev + s


def _tc_argmin(flat, embedding, a_norms, e_norms):
    m_total = flat.shape[0]
    return pl.pallas_call(
        _argmin_body,
        grid=(m_total // _BM,),
        in_specs=[
            pl.BlockSpec((_BM, _CODE_DIM), lambda m: (m, 0)),
            pl.BlockSpec((_NUM_CODES, _CODE_DIM), lambda m: (0, 0)),
            pl.BlockSpec((_BM, 1), lambda m: (m, 0)),
            pl.BlockSpec((1, _NUM_CODES), lambda m: (0, 0)),
        ],
        out_specs=[
            pl.BlockSpec((_BM, 1), lambda m: (m, 0)),
            pl.BlockSpec((1, 1), lambda m: (0, 0)),
        ],
        out_shape=[
            jax.ShapeDtypeStruct((m_total, 1), jnp.int32),
            jax.ShapeDtypeStruct((1, 1), jnp.float32),
        ],
    )(flat, embedding, a_norms, e_norms)


def _make_sc_gather(b_total):
    bpw = b_total // _NW
    nch = bpw // _GCH
    mesh = plsc.VectorSubcoreMesh(core_axis_name="c", subcore_axis_name="s")

    @functools.partial(
        pl.kernel, mesh=mesh,
        out_type=jax.ShapeDtypeStruct((b_total, _CODE_DIM), jnp.float32),
        scratch_types=[
            pltpu.VMEM((bpw,), jnp.int32),
            pltpu.VMEM((2, _GCH, _CODE_DIM), jnp.float32),
            pltpu.SemaphoreType.DMA,
            pltpu.SemaphoreType.DMA,
        ],
    )
    def gather(table_hbm, idx_hbm, out_hbm, idx_v, rows_v, sem0, sem1):
        wid = lax.axis_index("s") * _NC + lax.axis_index("c")
        base = wid * bpw
        pltpu.sync_copy(idx_hbm.at[pl.ds(base, bpw)], idx_v)
        sems = (sem0, sem1)
        copies = [None, None]
        for c in range(nch):
            sl = c % 2
            copies[sl] = pltpu.async_copy(
                table_hbm.at[idx_v.at[pl.ds(c * _GCH, _GCH)]],
                rows_v.at[sl], sems[sl])
            if c > 0:
                copies[(c - 1) % 2].wait()
                pltpu.sync_copy(rows_v.at[(c - 1) % 2],
                                out_hbm.at[pl.ds(base + (c - 1) * _GCH, _GCH)])
        copies[(nch - 1) % 2].wait()
        pltpu.sync_copy(rows_v.at[(nch - 1) % 2],
                        out_hbm.at[pl.ds(base + (nch - 1) * _GCH, _GCH)])

    return gather


def kernel(z, embedding):
    orig_shape = z.shape
    flat = z.reshape(-1, _CODE_DIM)
    m_total = flat.shape[0]
    a_norms = jnp.sum(flat * flat, axis=1, keepdims=True)
    e_norms = jnp.sum(embedding * embedding, axis=1).reshape(1, -1)
    idx2d, loss_sum = _tc_argmin(flat, embedding, a_norms, e_norms)
    indices = idx2d.reshape(-1)
    z_q_rows = _make_sc_gather(m_total)(embedding, indices)
    z_q_flat = z_q_rows.reshape(orig_shape)
    loss = (_COMMITMENT / (m_total * _CODE_DIM)) * loss_sum[0, 0]
    indices_out = indices.reshape(orig_shape[:-1] + (_SPLITS,))
    return (z_q_flat, loss, indices_out)


# speedup vs baseline: 1.2414x; 1.0940x over previous
import functools

import jax
import jax.numpy as jnp
from jax import lax
from jax.experimental import pallas as pl
from jax.experimental.pallas import tpu as pltpu
from jax.experimental.pallas import tpu_sc as plsc

_NUM_CODES = 8192
_CODE_DIM = 256
_SPLITS = 2
_COMMITMENT = 0.25

_BM = 2048
_BN = 2048
_NB = _NUM_CODES // _BN

_NC = 2
_NS = 16
_NW = _NC * _NS
_GCH = 128


def _argmin_body(flat_ref, emb_ref, an_ref, en_ref, idx_ref, loss_ref):
    m = pl.program_id(0)
    a = an_ref[...]
    flat = flat_ref[...]

    ids = lax.broadcasted_iota(jnp.int32, (_BM, _BN), 1).astype(jnp.float32)
    run_min = jnp.full((_BM, 1), jnp.inf, jnp.float32)
    run_arg = jnp.zeros((_BM, 1), jnp.float32)
    for i in range(_NB):
        eblk = emb_ref[pl.ds(i * _BN, _BN), :]
        b = lax.dot_general(flat, eblk, (((1,), (1,)), ((), ())),
                            preferred_element_type=jnp.float32)
        c = en_ref[:, pl.ds(i * _BN, _BN)]
        dist = (a - 2.0 * b) + c
        lmin = jnp.min(dist, axis=1, keepdims=True)
        larg = jnp.min(jnp.where(dist == lmin, ids, jnp.float32(2**24)),
                       axis=1, keepdims=True) + jnp.float32(i * _BN)
        upd = lmin < run_min
        run_min = jnp.where(upd, lmin, run_min)
        run_arg = jnp.where(upd, larg, run_arg)
    idx_ref[...] = run_arg.astype(jnp.int32)
    s = jnp.sum(run_min, keepdims=True)
    prev = jnp.where(m == 0, jnp.zeros((1, 1), jnp.float32), loss_ref[...])
    loss_ref[...] = prev + s


def _tc_argmin(flat, embedding, a_norms, e_norms):
    m_total = flat.shape[0]
    return pl.pallas_call(
        _argmin_body,
        grid=(m_total // _BM,),
        in_specs=[
            pl.BlockSpec((_BM, _CODE_DIM), lambda m: (m, 0)),
            pl.BlockSpec((_NUM_CODES, _CODE_DIM), lambda m: (0, 0)),
            pl.BlockSpec((_BM, 1), lambda m: (m, 0)),
            pl.BlockSpec((1, _NUM_CODES), lambda m: (0, 0)),
        ],
        out_specs=[
            pl.BlockSpec((_BM, 1), lambda m: (m, 0)),
            pl.BlockSpec((1, 1), lambda m: (0, 0)),
        ],
        out_shape=[
            jax.ShapeDtypeStruct((m_total, 1), jnp.int32),
            jax.ShapeDtypeStruct((1, 1), jnp.float32),
        ],
    )(flat, embedding, a_norms, e_norms)


def _make_sc_gather(b_total):
    bpw = b_total // _NW
    nch = bpw // _GCH
    mesh = plsc.VectorSubcoreMesh(core_axis_name="c", subcore_axis_name="s")

    @functools.partial(
        pl.kernel, mesh=mesh,
        out_type=jax.ShapeDtypeStruct((b_total, _CODE_DIM), jnp.float32),
        scratch_types=[
            pltpu.VMEM((bpw,), jnp.int32),
            pltpu.VMEM((2, _GCH, _CODE_DIM), jnp.float32),
            pltpu.SemaphoreType.DMA,
            pltpu.SemaphoreType.DMA,
        ],
    )
    def gather(table_hbm, idx_hbm, out_hbm, idx_v, rows_v, sem0, sem1):
        wid = lax.axis_index("s") * _NC + lax.axis_index("c")
        base = wid * bpw
        pltpu.sync_copy(idx_hbm.at[pl.ds(base, bpw)], idx_v)
        sems = (sem0, sem1)
        copies = [None, None]
        for c in range(nch):
            sl = c % 2
            copies[sl] = pltpu.async_copy(
                table_hbm.at[idx_v.at[pl.ds(c * _GCH, _GCH)]],
                rows_v.at[sl], sems[sl])
            if c > 0:
                copies[(c - 1) % 2].wait()
                pltpu.sync_copy(rows_v.at[(c - 1) % 2],
                                out_hbm.at[pl.ds(base + (c - 1) * _GCH, _GCH)])
        copies[(nch - 1) % 2].wait()
        pltpu.sync_copy(rows_v.at[(nch - 1) % 2],
                        out_hbm.at[pl.ds(base + (nch - 1) * _GCH, _GCH)])

    return gather


def kernel(z, embedding):
    orig_shape = z.shape
    flat = z.reshape(-1, _CODE_DIM)
    m_total = flat.shape[0]
    a_norms = jnp.sum(flat * flat, axis=1, keepdims=True)
    e_norms = jnp.sum(embedding * embedding, axis=1).reshape(1, -1)
    idx2d, loss_sum = _tc_argmin(flat, embedding, a_norms, e_norms)
    indices = idx2d.reshape(-1)
    z_q_rows = _make_sc_gather(m_total)(embedding, indices)
    z_q_flat = z_q_rows.reshape(orig_shape)
    loss = (_COMMITMENT / (m_total * _CODE_DIM)) * loss_sum[0, 0]
    indices_out = indices.reshape(orig_shape[:-1] + (_SPLITS,))
    return (z_q_flat, loss, indices_out)


# speedup vs baseline: 1.2806x; 1.0316x over previous
import functools

import jax
import jax.numpy as jnp
from jax import lax
from jax.experimental import pallas as pl
from jax.experimental.pallas import tpu as pltpu
from jax.experimental.pallas import tpu_sc as plsc

_NUM_CODES = 8192
_CODE_DIM = 256
_SPLITS = 2
_COMMITMENT = 0.25

_BM = 2048
_BN = 4096
_NB = _NUM_CODES // _BN

_NC = 2
_NS = 16
_NW = _NC * _NS
_GCH = 128


def _argmin_body(flat_ref, emb_ref, an_ref, en_ref, idx_ref, loss_ref):
    m = pl.program_id(0)
    a = an_ref[...]
    flat = flat_ref[...]

    ids = lax.broadcasted_iota(jnp.int32, (_BM, _BN), 1).astype(jnp.float32)
    run_min = jnp.full((_BM, 1), jnp.inf, jnp.float32)
    run_arg = jnp.zeros((_BM, 1), jnp.float32)
    for i in range(_NB):
        eblk = emb_ref[pl.ds(i * _BN, _BN), :]
        b = lax.dot_general(flat, eblk, (((1,), (1,)), ((), ())),
                            preferred_element_type=jnp.float32)
        c = en_ref[:, pl.ds(i * _BN, _BN)]
        dist = (a - 2.0 * b) + c
        lmin = jnp.min(dist, axis=1, keepdims=True)
        larg = jnp.min(jnp.where(dist == lmin, ids, jnp.float32(2**24)),
                       axis=1, keepdims=True) + jnp.float32(i * _BN)
        upd = lmin < run_min
        run_min = jnp.where(upd, lmin, run_min)
        run_arg = jnp.where(upd, larg, run_arg)
    idx_ref[...] = run_arg.astype(jnp.int32)
    s = jnp.sum(run_min, keepdims=True)
    prev = jnp.where(m == 0, jnp.zeros((1, 1), jnp.float32), loss_ref[...])
    loss_ref[...] = prev + s


def _tc_argmin(flat, embedding, a_norms, e_norms):
    m_total = flat.shape[0]
    return pl.pallas_call(
        _argmin_body,
        grid=(m_total // _BM,),
        in_specs=[
            pl.BlockSpec((_BM, _CODE_DIM), lambda m: (m, 0)),
            pl.BlockSpec((_NUM_CODES, _CODE_DIM), lambda m: (0, 0)),
            pl.BlockSpec((_BM, 1), lambda m: (m, 0)),
            pl.BlockSpec((1, _NUM_CODES), lambda m: (0, 0)),
        ],
        out_specs=[
            pl.BlockSpec((_BM, 1), lambda m: (m, 0)),
            pl.BlockSpec((1, 1), lambda m: (0, 0)),
        ],
        out_shape=[
            jax.ShapeDtypeStruct((m_total, 1), jnp.int32),
            jax.ShapeDtypeStruct((1, 1), jnp.float32),
        ],
    )(flat, embedding, a_norms, e_norms)


def _make_sc_gather(b_total):
    bpw = b_total // _NW
    nch = bpw // _GCH
    mesh = plsc.VectorSubcoreMesh(core_axis_name="c", subcore_axis_name="s")

    @functools.partial(
        pl.kernel, mesh=mesh,
        out_type=jax.ShapeDtypeStruct((b_total, _CODE_DIM), jnp.float32),
        scratch_types=[
            pltpu.VMEM((bpw,), jnp.int32),
            pltpu.VMEM((2, _GCH, _CODE_DIM), jnp.float32),
            pltpu.SemaphoreType.DMA,
            pltpu.SemaphoreType.DMA,
        ],
    )
    def gather(table_hbm, idx_hbm, out_hbm, idx_v, rows_v, sem0, sem1):
        wid = lax.axis_index("s") * _NC + lax.axis_index("c")
        base = wid * bpw
        pltpu.sync_copy(idx_hbm.at[pl.ds(base, bpw)], idx_v)
        sems = (sem0, sem1)
        copies = [None, None]
        for c in range(nch):
            sl = c % 2
            copies[sl] = pltpu.async_copy(
                table_hbm.at[idx_v.at[pl.ds(c * _GCH, _GCH)]],
                rows_v.at[sl], sems[sl])
            if c > 0:
                copies[(c - 1) % 2].wait()
                pltpu.sync_copy(rows_v.at[(c - 1) % 2],
                                out_hbm.at[pl.ds(base + (c - 1) * _GCH, _GCH)])
        copies[(nch - 1) % 2].wait()
        pltpu.sync_copy(rows_v.at[(nch - 1) % 2],
                        out_hbm.at[pl.ds(base + (nch - 1) * _GCH, _GCH)])

    return gather


def kernel(z, embedding):
    orig_shape = z.shape
    flat = z.reshape(-1, _CODE_DIM)
    m_total = flat.shape[0]
    a_norms = jnp.sum(flat * flat, axis=1, keepdims=True)
    e_norms = jnp.sum(embedding * embedding, axis=1).reshape(1, -1)
    idx2d, loss_sum = _tc_argmin(flat, embedding, a_norms, e_norms)
    indices = idx2d.reshape(-1)
    z_q_rows = _make_sc_gather(m_total)(embedding, indices)
    z_q_flat = z_q_rows.reshape(orig_shape)
    loss = (_COMMITMENT / (m_total * _CODE_DIM)) * loss_sum[0, 0]
    indices_out = indices.reshape(orig_shape[:-1] + (_SPLITS,))
    return (z_q_flat, loss, indices_out)


# speedup vs baseline: 1.2811x; 1.0004x over previous
import functools

import jax
import jax.numpy as jnp
from jax import lax
from jax.experimental import pallas as pl
from jax.experimental.pallas import tpu as pltpu
from jax.experimental.pallas import tpu_sc as plsc

_NUM_CODES = 8192
_CODE_DIM = 256
_SPLITS = 2
_COMMITMENT = 0.25

_BM = 2048
_BN = 4096
_NB = _NUM_CODES // _BN

_NC = 2
_NS = 16
_NW = _NC * _NS
_GCH = 64


def _argmin_body(flat_ref, emb_ref, an_ref, en_ref, idx_ref, loss_ref):
    m = pl.program_id(0)
    a = an_ref[...]
    flat = flat_ref[...]

    ids = lax.broadcasted_iota(jnp.int32, (_BM, _BN), 1).astype(jnp.float32)
    run_min = jnp.full((_BM, 1), jnp.inf, jnp.float32)
    run_arg = jnp.zeros((_BM, 1), jnp.float32)
    for i in range(_NB):
        eblk = emb_ref[pl.ds(i * _BN, _BN), :]
        b = lax.dot_general(flat, eblk, (((1,), (1,)), ((), ())),
                            preferred_element_type=jnp.float32)
        c = en_ref[:, pl.ds(i * _BN, _BN)]
        dist = (a - 2.0 * b) + c
        lmin = jnp.min(dist, axis=1, keepdims=True)
        larg = jnp.min(jnp.where(dist == lmin, ids, jnp.float32(2**24)),
                       axis=1, keepdims=True) + jnp.float32(i * _BN)
        upd = lmin < run_min
        run_min = jnp.where(upd, lmin, run_min)
        run_arg = jnp.where(upd, larg, run_arg)
    idx_ref[...] = run_arg.astype(jnp.int32)
    s = jnp.sum(run_min, keepdims=True)
    prev = jnp.where(m == 0, jnp.zeros((1, 1), jnp.float32), loss_ref[...])
    loss_ref[...] = prev + s


def _tc_argmin(flat, embedding, a_norms, e_norms):
    m_total = flat.shape[0]
    return pl.pallas_call(
        _argmin_body,
        grid=(m_total // _BM,),
        in_specs=[
            pl.BlockSpec((_BM, _CODE_DIM), lambda m: (m, 0)),
            pl.BlockSpec((_NUM_CODES, _CODE_DIM), lambda m: (0, 0)),
            pl.BlockSpec((_BM, 1), lambda m: (m, 0)),
            pl.BlockSpec((1, _NUM_CODES), lambda m: (0, 0)),
        ],
        out_specs=[
            pl.BlockSpec((_BM, 1), lambda m: (m, 0)),
            pl.BlockSpec((1, 1), lambda m: (0, 0)),
        ],
        out_shape=[
            jax.ShapeDtypeStruct((m_total, 1), jnp.int32),
            jax.ShapeDtypeStruct((1, 1), jnp.float32),
        ],
    )(flat, embedding, a_norms, e_norms)


def _make_sc_gather(b_total):
    bpw = b_total // _NW
    nch = bpw // _GCH
    nbuf = 4
    mesh = plsc.VectorSubcoreMesh(core_axis_name="c", subcore_axis_name="s")

    @functools.partial(
        pl.kernel, mesh=mesh,
        out_type=jax.ShapeDtypeStruct((b_total, _CODE_DIM), jnp.float32),
        scratch_types=(
            [pltpu.VMEM((bpw,), jnp.int32),
             pltpu.VMEM((nbuf, _GCH, _CODE_DIM), jnp.float32)]
            + [pltpu.SemaphoreType.DMA] * (2 * nbuf)
        ),
    )
    def gather(table_hbm, idx_hbm, out_hbm, idx_v, rows_v, *sems):
        gsem, ssem = sems[:nbuf], sems[nbuf:]
        wid = lax.axis_index("s") * _NC + lax.axis_index("c")
        base = wid * bpw
        pltpu.sync_copy(idx_hbm.at[pl.ds(base, bpw)], idx_v)
        gs, ss = [None] * nch, [None] * nch

        def _drain(c):
            bi = c % nbuf
            gs[c].wait()
            ss[c] = pltpu.async_copy(
                rows_v.at[bi], out_hbm.at[pl.ds(base + c * _GCH, _GCH)],
                ssem[bi])

        for c in range(nch):
            bi = c % nbuf
            if c >= nbuf:
                ss[c - nbuf].wait()
            gs[c] = pltpu.async_copy(
                table_hbm.at[idx_v.at[pl.ds(c * _GCH, _GCH)]],
                rows_v.at[bi], gsem[bi])
            if c >= nbuf - 1:
                _drain(c - (nbuf - 1))
        for c in range(nch - (nbuf - 1), nch):
            _drain(c)
        for c in range(nch - nbuf, nch):
            if c >= 0:
                ss[c].wait()

    return gather


def kernel(z, embedding):
    orig_shape = z.shape
    flat = z.reshape(-1, _CODE_DIM)
    m_total = flat.shape[0]
    a_norms = jnp.sum(flat * flat, axis=1, keepdims=True)
    e_norms = jnp.sum(embedding * embedding, axis=1).reshape(1, -1)
    idx2d, loss_sum = _tc_argmin(flat, embedding, a_norms, e_norms)
    indices = idx2d.reshape(-1)
    z_q_rows = _make_sc_gather(m_total)(embedding, indices)
    z_q_flat = z_q_rows.reshape(orig_shape)
    loss = (_COMMITMENT / (m_total * _CODE_DIM)) * loss_sum[0, 0]
    indices_out = indices.reshape(orig_shape[:-1] + (_SPLITS,))
    return (z_q_flat, loss, indices_out)
